# baseline (speedup 1.0000x reference)
import functools

import jax
import jax.numpy as jnp
from jax import lax
from jax.experimental import pallas as pl
from jax.experimental.pallas import tpu as pltpu
from jax.experimental.pallas import tpu_sc as plsc

NC = 2
NS = 16
L = 16
NW = NC * NS
N = 16 * 1024 * 1024
PER_W = N // NW
CHUNK = 16384
NCHUNK = PER_W // CHUNK
WORDS = CHUNK // 4
TOT = 1048576.0

_mesh = plsc.VectorSubcoreMesh(core_axis_name="c", subcore_axis_name="s")
_params = pltpu.CompilerParams(needs_layout_passes=False)


@functools.partial(
    pl.kernel,
    out_type=(
        jax.ShapeDtypeStruct((NW * 16,), jnp.int32),
        jax.ShapeDtypeStruct((N // 4,), jnp.int32),
    ),
    mesh=_mesh,
    scratch_types=[
        pltpu.VMEM((CHUNK,), jnp.float32),
        pltpu.VMEM((CHUNK,), jnp.float32),
        pltpu.VMEM((WORDS,), jnp.int32),
        pltpu.VMEM((256,), jnp.int32),
    ],
    compiler_params=_params,
)
def _pass1(pred_hbm, target_hbm, hist_hbm, bins_hbm, pbuf, tbuf, wbuf, hbuf):
    wid = lax.axis_index("s") * NC + lax.axis_index("c")
    base = wid * PER_W
    lane16 = lax.iota(jnp.int32, L) * 16
    zeros = jnp.zeros((L,), jnp.int32)
    ones = jnp.ones((L,), jnp.int32)
    for i in range(16):
        hbuf[pl.ds(i * L, L)] = zeros

    def chunk_body(ci, carry):
        off = pl.multiple_of(base + ci * CHUNK, CHUNK)
        pltpu.sync_copy(pred_hbm.at[pl.ds(off, CHUNK)], pbuf)
        pltpu.sync_copy(target_hbm.at[pl.ds(off, CHUNK)], tbuf)

        def vec_body(j, c2):
            e = j * 64
            bs = []
            for u in range(4):
                p = pbuf[pl.ds(e + u * L, L)]
                t = tbuf[pl.ds(e + u * L, L)]
                g = jnp.abs(p - t)
                b = (g * 10.0).astype(jnp.int32)
                plsc.addupdate_scatter(hbuf, [lane16 + b], ones)
                bs.append(b)
            word = bs[0] | (bs[1] << 8) | (bs[2] << 16) | (bs[3] << 24)
            wbuf[pl.ds(j * L, L)] = word
            return c2

        lax.fori_loop(0, WORDS // L, vec_body, 0)
        woff = pl.multiple_of(wid * (PER_W // 4) + ci * WORDS, WORDS)
        pltpu.sync_copy(wbuf, bins_hbm.at[pl.ds(woff, WORDS)])
        return carry

    lax.fori_loop(0, NCHUNK, chunk_body, 0)

    acc = hbuf[pl.ds(0, L)]
    for i in range(1, 16):
        acc = acc + hbuf[pl.ds(i * L, L)]
    wbuf[pl.ds(0, L)] = acc
    pltpu.sync_copy(wbuf.at[pl.ds(0, L)], hist_hbm.at[pl.ds(wid * L, L)])


@functools.partial(
    pl.kernel,
    out_type=jax.ShapeDtypeStruct((N,), jnp.float32),
    mesh=_mesh,
    scratch_types=[
        pltpu.VMEM((NW * 16,), jnp.int32),
        pltpu.VMEM((WORDS,), jnp.int32),
        pltpu.VMEM((CHUNK,), jnp.float32),
        pltpu.VMEM((16,), jnp.float32),
    ],
    compiler_params=_params,
)
def _pass2(hist_hbm, bins_hbm, out_hbm, histbuf, wordbuf, obuf, tabbuf):
    wid = lax.axis_index("s") * NC + lax.axis_index("c")
    pltpu.sync_copy(hist_hbm, histbuf)
    counts = histbuf[pl.ds(0, L)]
    for wdx in range(1, NW):
        counts = counts + histbuf[pl.ds(wdx * L, L)]
    cf = counts.astype(jnp.float32)
    nz = counts > 0
    n = plsc.all_reduce_population_count(nz)
    nf = jnp.maximum(n.astype(jnp.float32), 1.0)
    acc = jnp.maximum(cf * 0.9, 1e-12)
    tabbuf[...] = jnp.where(nz, TOT / acc, 0.0) / nf

    base = wid * PER_W

    def chunk_body(ci, carry):
        off = pl.multiple_of(base + ci * CHUNK, CHUNK)
        woff = pl.multiple_of(wid * (PER_W // 4) + ci * WORDS, WORDS)
        pltpu.sync_copy(bins_hbm.at[pl.ds(woff, WORDS)], wordbuf)

        def vec_body(j, c2):
            word = wordbuf[pl.ds(j * L, L)]
            e = j * 64
            parts = (
                word & 255,
                (word >> 8) & 255,
                (word >> 16) & 255,
                lax.shift_right_logical(word, 24),
            )
            for u in range(4):
                obuf[pl.ds(e + u * L, L)] = plsc.load_gather(tabbuf, [parts[u]])
            return c2

        lax.fori_loop(0, WORDS // L, vec_body, 0)
        pltpu.sync_copy(obuf, out_hbm.at[pl.ds(off, CHUNK)])
        return carry

    lax.fori_loop(0, NCHUNK, chunk_body, 0)


def kernel(pred, target):
    p = pred.reshape(-1)
    t = target.astype(jnp.float32).reshape(-1)
    hist, bins = _pass1(p, t)
    w = _pass2(hist, bins)
    return w.reshape(pred.shape)


# speedup vs baseline: 1.2563x; 1.2563x over previous
import functools

import jax
import jax.numpy as jnp
from jax import lax
from jax.experimental import pallas as pl
from jax.experimental.pallas import tpu as pltpu
from jax.experimental.pallas import tpu_sc as plsc

NC = 2
NS = 16
L = 16
NW = NC * NS
B, R, C = 16, 1024, 1024
N = B * R * C
ROWS_W = (B * R) // NW
RCHUNK = 16
NCHUNK = ROWS_W // RCHUNK
CHUNK = RCHUNK * C
WORDS = CHUNK // 4
TOT = float(R * C)

_mesh = plsc.VectorSubcoreMesh(core_axis_name="c", subcore_axis_name="s")
_params = pltpu.CompilerParams(
    needs_layout_passes=False, use_tc_tiling_on_sc=True)


@functools.partial(
    pl.kernel,
    out_type=(
        jax.ShapeDtypeStruct((NW * 16,), jnp.int32),
        jax.ShapeDtypeStruct((N // 4,), jnp.int32),
    ),
    mesh=_mesh,
    scratch_types=[
        pltpu.VMEM((RCHUNK, C), jnp.float32),
        pltpu.VMEM((RCHUNK, C), jnp.float32),
        pltpu.VMEM((WORDS,), jnp.int32),
        pltpu.VMEM((256,), jnp.int32),
    ],
    compiler_params=_params,
)
def _pass1(pred_hbm, target_hbm, hist_hbm, bins_hbm, pbuf, tbuf, wbuf, hbuf):
    wid = lax.axis_index("s") * NC + lax.axis_index("c")
    b = wid // 2
    rbase = (wid % 2) * ROWS_W
    lane16 = lax.iota(jnp.int32, L) * 16
    zeros = jnp.zeros((L,), jnp.int32)
    ones = jnp.ones((L,), jnp.int32)
    for i in range(16):
        hbuf[pl.ds(i * L, L)] = zeros

    def chunk_body(ci, carry):
        r0 = rbase + ci * RCHUNK
        pltpu.sync_copy(pred_hbm.at[b, pl.ds(r0, RCHUNK), :], pbuf)
        pltpu.sync_copy(target_hbm.at[b, pl.ds(r0, RCHUNK), :], tbuf)

        def vec_body(j, c2):
            c0 = j * 64
            for r in range(RCHUNK):
                bs = []
                for u in range(4):
                    p = pbuf[r, pl.ds(c0 + u * L, L)]
                    t = tbuf[r, pl.ds(c0 + u * L, L)]
                    g = jnp.abs(p - t)
                    bv = (g * 10.0).astype(jnp.int32)
                    plsc.addupdate_scatter(hbuf, [lane16 + bv], ones)
                    bs.append(bv)
                word = bs[0] | (bs[1] << 8) | (bs[2] << 16) | (bs[3] << 24)
                wbuf[pl.ds((r * 16 + j) * L, L)] = word
            return c2

        lax.fori_loop(0, 16, vec_body, 0)
        woff = pl.multiple_of((wid * NCHUNK + ci) * WORDS, WORDS)
        pltpu.sync_copy(wbuf, bins_hbm.at[pl.ds(woff, WORDS)])
        return carry

    lax.fori_loop(0, NCHUNK, chunk_body, 0)

    acc = hbuf[pl.ds(0, L)]
    for i in range(1, 16):
        acc = acc + hbuf[pl.ds(i * L, L)]
    wbuf[pl.ds(0, L)] = acc
    pltpu.sync_copy(wbuf.at[pl.ds(0, L)], hist_hbm.at[pl.ds(wid * L, L)])


@functools.partial(
    pl.kernel,
    out_type=jax.ShapeDtypeStruct((B, R, C), jnp.float32),
    mesh=_mesh,
    scratch_types=[
        pltpu.VMEM((NW * 16,), jnp.int32),
        pltpu.VMEM((WORDS,), jnp.int32),
        pltpu.VMEM((RCHUNK, C), jnp.float32),
        pltpu.VMEM((16,), jnp.float32),
    ],
    compiler_params=_params,
)
def _pass2(hist_hbm, bins_hbm, out_hbm, histbuf, wordbuf, obuf, tabbuf):
    wid = lax.axis_index("s") * NC + lax.axis_index("c")
    b = wid // 2
    rbase = (wid % 2) * ROWS_W
    pltpu.sync_copy(hist_hbm, histbuf)
    counts = histbuf[pl.ds(0, L)]
    for wdx in range(1, NW):
        counts = counts + histbuf[pl.ds(wdx * L, L)]
    cf = counts.astype(jnp.float32)
    nz = counts > 0
    n = plsc.all_reduce_population_count(nz)
    nf = jnp.maximum(n.astype(jnp.float32), 1.0)
    acc = jnp.maximum(cf * 0.9, 1e-12)
    tabbuf[...] = jnp.where(nz, TOT / acc, 0.0) / nf

    def chunk_body(ci, carry):
        r0 = rbase + ci * RCHUNK
        woff = pl.multiple_of((wid * NCHUNK + ci) * WORDS, WORDS)
        pltpu.sync_copy(bins_hbm.at[pl.ds(woff, WORDS)], wordbuf)

        def vec_body(j, c2):
            c0 = j * 64
            for r in range(RCHUNK):
                word = wordbuf[pl.ds((r * 16 + j) * L, L)]
                parts = (
                    word & 255,
                    (word >> 8) & 255,
                    (word >> 16) & 255,
                    lax.shift_right_logical(word, 24),
                )
                for u in range(4):
                    obuf[r, pl.ds(c0 + u * L, L)] = plsc.load_gather(
                        tabbuf, [parts[u]])
            return c2

        lax.fori_loop(0, 16, vec_body, 0)
        pltpu.sync_copy(obuf, out_hbm.at[b, pl.ds(r0, RCHUNK), :])
        return carry

    lax.fori_loop(0, NCHUNK, chunk_body, 0)


def kernel(pred, target):
    t = target.astype(jnp.float32)
    hist, bins = _pass1(pred, t)
    return _pass2(hist, bins)


# speedup vs baseline: 1.5245x; 1.2135x over previous
import functools

import jax
import jax.numpy as jnp
from jax import lax
from jax.experimental import pallas as pl
from jax.experimental.pallas import tpu as pltpu
from jax.experimental.pallas import tpu_sc as plsc

NC = 2
NS = 16
L = 16
NW = NC * NS
B, R, C = 16, 1024, 1024
N = B * R * C
ROWS_W = (B * R) // NW
RCHUNK = 16
NCHUNK = ROWS_W // RCHUNK
CHUNK = RCHUNK * C
WORDS = CHUNK // 4
TOT = float(R * C)

_mesh = plsc.VectorSubcoreMesh(core_axis_name="c", subcore_axis_name="s")
_params = pltpu.CompilerParams(
    needs_layout_passes=False, use_tc_tiling_on_sc=True)


@functools.partial(
    pl.kernel,
    out_type=(
        jax.ShapeDtypeStruct((NW * 16,), jnp.int32),
        jax.ShapeDtypeStruct((N // 4,), jnp.int32),
    ),
    mesh=_mesh,
    scratch_types=[
        pltpu.VMEM((RCHUNK, C), jnp.float32),
        pltpu.VMEM((RCHUNK, C), jnp.float32),
        pltpu.VMEM((RCHUNK, C), jnp.float32),
        pltpu.VMEM((RCHUNK, C), jnp.float32),
        pltpu.VMEM((WORDS,), jnp.int32),
        pltpu.VMEM((WORDS,), jnp.int32),
        pltpu.VMEM((8 * 256,), jnp.int32),
        pltpu.SemaphoreType.DMA,
        pltpu.SemaphoreType.DMA,
        pltpu.SemaphoreType.DMA,
        pltpu.SemaphoreType.DMA,
    ],
    compiler_params=_params,
)
def _pass1(pred_hbm, target_hbm, hist_hbm, bins_hbm,
           pbuf0, pbuf1, tbuf0, tbuf1, wbuf0, wbuf1, hbuf,
           isem0, isem1, osem0, osem1):
    wid = lax.axis_index("s") * NC + lax.axis_index("c")
    b = wid // 2
    rbase = (wid % 2) * ROWS_W
    pbufs, tbufs, wbufs = (pbuf0, pbuf1), (tbuf0, tbuf1), (wbuf0, wbuf1)
    isems, osems = (isem0, isem1), (osem0, osem1)
    lane16 = lax.iota(jnp.int32, L) * 16
    zeros = jnp.zeros((L,), jnp.int32)
    ones = jnp.ones((L,), jnp.int32)
    for i in range(8 * 16):
        hbuf[pl.ds(i * L, L)] = zeros

    def start_in(ci, s):
        r0 = rbase + ci * RCHUNK
        pltpu.async_copy(pred_hbm.at[b, pl.ds(r0, RCHUNK), :], pbufs[s],
                         isems[s])
        pltpu.async_copy(target_hbm.at[b, pl.ds(r0, RCHUNK), :], tbufs[s],
                         isems[s])

    def wait_in(s):
        pltpu.make_async_copy(
            pred_hbm.at[b, pl.ds(rbase, RCHUNK), :], pbufs[s], isems[s]).wait()
        pltpu.make_async_copy(
            target_hbm.at[b, pl.ds(rbase, RCHUNK), :], tbufs[s],
            isems[s]).wait()

    def wait_out(s):
        pltpu.make_async_copy(
            bins_hbm.at[pl.ds(0, WORDS)], wbufs[s], osems[s]).wait()

    start_in(0, 0)
    start_in(1, 1)

    @pl.loop(0, NCHUNK // 2)
    def ring(k):
        for s in range(2):
            ci = k * 2 + s
            wait_in(s)

            @pl.when(k > 0)
            def _():
                wait_out(s)

            pb, tb, wb = pbufs[s], tbufs[s], wbufs[s]

            @pl.loop(0, 16)
            def vec_body(j):
                c0 = j * 64
                for r in range(RCHUNK):
                    reg = (r % 2) * 4 * 256
                    bs = []
                    for u in range(4):
                        p = pb[r, pl.ds(c0 + u * L, L)]
                        t = tb[r, pl.ds(c0 + u * L, L)]
                        g = jnp.abs(p - t)
                        bv = (g * 10.0).astype(jnp.int32)
                        plsc.addupdate_scatter(
                            hbuf, [lane16 + (bv + (reg + u * 256))], ones)
                        bs.append(bv)
                    word = bs[0] | (bs[1] << 8) | (bs[2] << 16) | (bs[3] << 24)
                    wb[pl.ds((r * 16 + j) * L, L)] = word

            woff = pl.multiple_of((wid * NCHUNK + ci) * WORDS, WORDS)
            pltpu.async_copy(wb, bins_hbm.at[pl.ds(woff, WORDS)], osems[s])

            @pl.when(ci + 2 < NCHUNK)
            def _():
                start_in(ci + 2, s)

    wait_out(0)
    wait_out(1)

    acc = hbuf[pl.ds(0, L)]
    for i in range(1, 8 * 16):
        acc = acc + hbuf[pl.ds(i * L, L)]
    wbuf0[pl.ds(0, L)] = acc
    pltpu.sync_copy(wbuf0.at[pl.ds(0, L)], hist_hbm.at[pl.ds(wid * L, L)])


@functools.partial(
    pl.kernel,
    out_type=jax.ShapeDtypeStruct((B, R, C), jnp.float32),
    mesh=_mesh,
    scratch_types=[
        pltpu.VMEM((NW * 16,), jnp.int32),
        pltpu.VMEM((WORDS,), jnp.int32),
        pltpu.VMEM((WORDS,), jnp.int32),
        pltpu.VMEM((RCHUNK, C), jnp.float32),
        pltpu.VMEM((RCHUNK, C), jnp.float32),
        pltpu.VMEM((16,), jnp.float32),
        pltpu.SemaphoreType.DMA,
        pltpu.SemaphoreType.DMA,
        pltpu.SemaphoreType.DMA,
        pltpu.SemaphoreType.DMA,
    ],
    compiler_params=_params,
)
def _pass2(hist_hbm, bins_hbm, out_hbm, histbuf, wordbuf0, wordbuf1,
           obuf0, obuf1, tabbuf, isem0, isem1, osem0, osem1):
    wid = lax.axis_index("s") * NC + lax.axis_index("c")
    b = wid // 2
    rbase = (wid % 2) * ROWS_W
    wordbufs, obufs = (wordbuf0, wordbuf1), (obuf0, obuf1)
    isems, osems = (isem0, isem1), (osem0, osem1)

    pltpu.sync_copy(hist_hbm, histbuf)
    counts = histbuf[pl.ds(0, L)]
    for wdx in range(1, NW):
        counts = counts + histbuf[pl.ds(wdx * L, L)]
    cf = counts.astype(jnp.float32)
    nz = counts > 0
    n = plsc.all_reduce_population_count(nz)
    nf = jnp.maximum(n.astype(jnp.float32), 1.0)
    acc = jnp.maximum(cf * 0.9, 1e-12)
    tabbuf[...] = jnp.where(nz, TOT / acc, 0.0) / nf

    def start_in(ci, s):
        woff = pl.multiple_of((wid * NCHUNK + ci) * WORDS, WORDS)
        pltpu.async_copy(bins_hbm.at[pl.ds(woff, WORDS)], wordbufs[s],
                         isems[s])

    def wait_in(s):
        pltpu.make_async_copy(
            bins_hbm.at[pl.ds(0, WORDS)], wordbufs[s], isems[s]).wait()

    def wait_out(s):
        pltpu.make_async_copy(
            out_hbm.at[b, pl.ds(rbase, RCHUNK), :], obufs[s], osems[s]).wait()

    start_in(0, 0)
    start_in(1, 1)

    @pl.loop(0, NCHUNK // 2)
    def ring(k):
        for s in range(2):
            ci = k * 2 + s
            wait_in(s)

            @pl.when(k > 0)
            def _():
                wait_out(s)

            wb, ob = wordbufs[s], obufs[s]

            @pl.loop(0, 16)
            def vec_body(j):
                c0 = j * 64
                for r in range(RCHUNK):
                    word = wb[pl.ds((r * 16 + j) * L, L)]
                    parts = (
                        word & 255,
                        (word >> 8) & 255,
                        (word >> 16) & 255,
                        lax.shift_right_logical(word, 24),
                    )
                    for u in range(4):
                        ob[r, pl.ds(c0 + u * L, L)] = plsc.load_gather(
                            tabbuf, [parts[u]])

            r0 = rbase + ci * RCHUNK
            pltpu.async_copy(ob, out_hbm.at[b, pl.ds(r0, RCHUNK), :], osems[s])

            @pl.when(ci + 2 < NCHUNK)
            def _():
                start_in(ci + 2, s)

    wait_out(0)
    wait_out(1)


def kernel(pred, target):
    t = target.astype(jnp.float32)
    hist, bins = _pass1(pred, t)
    return _pass2(hist, bins)


# speedup vs baseline: 5.3759x; 3.5263x over previous
import functools

import jax
import jax.numpy as jnp
from jax import lax
from jax.experimental import pallas as pl
from jax.experimental.pallas import tpu as pltpu
from jax.experimental.pallas import tpu_sc as plsc

NC = 2
NS = 16
L = 16
NW = NC * NS
B, R, C = 16, 1024, 1024
N = B * R * C
ROWS_W = (B * R) // NW
RCHUNK = 16
NCHUNK = ROWS_W // RCHUNK
CHUNK = RCHUNK * C
WORDS = CHUNK // 4
TOT = float(R * C)

_mesh = plsc.VectorSubcoreMesh(core_axis_name="c", subcore_axis_name="s")
_params = pltpu.CompilerParams(
    needs_layout_passes=False, use_tc_tiling_on_sc=True)

_HSCRATCH = [pltpu.VMEM((256,), jnp.int32) for _ in range(8)]


@functools.partial(
    pl.kernel,
    out_type=(
        jax.ShapeDtypeStruct((NW * 16,), jnp.int32),
        jax.ShapeDtypeStruct((N // 4,), jnp.int32),
    ),
    mesh=_mesh,
    scratch_types=[
        pltpu.VMEM((RCHUNK, C), jnp.float32),
        pltpu.VMEM((RCHUNK, C), jnp.float32),
        pltpu.VMEM((RCHUNK, C), jnp.float32),
        pltpu.VMEM((RCHUNK, C), jnp.float32),
        pltpu.VMEM((WORDS,), jnp.int32),
        pltpu.VMEM((WORDS,), jnp.int32),
    ] + _HSCRATCH + [
        pltpu.SemaphoreType.DMA,
        pltpu.SemaphoreType.DMA,
        pltpu.SemaphoreType.DMA,
        pltpu.SemaphoreType.DMA,
    ],
    compiler_params=_params,
)
def _pass1(pred_hbm, target_hbm, hist_hbm, bins_hbm,
           pbuf0, pbuf1, tbuf0, tbuf1, wbuf0, wbuf1,
           h0, h1, h2, h3, h4, h5, h6, h7,
           isem0, isem1, osem0, osem1):
    wid = lax.axis_index("s") * NC + lax.axis_index("c")
    b = wid // 2
    rbase = (wid % 2) * ROWS_W
    pbufs, tbufs, wbufs = (pbuf0, pbuf1), (tbuf0, tbuf1), (wbuf0, wbuf1)
    hbufs = (h0, h1, h2, h3, h4, h5, h6, h7)
    isems, osems = (isem0, isem1), (osem0, osem1)
    lane16 = lax.iota(jnp.int32, L) * 16
    zeros = jnp.zeros((L,), jnp.int32)
    ones = jnp.ones((L,), jnp.int32)
    for m in range(8):
        for i in range(16):
            hbufs[m][pl.ds(i * L, L)] = zeros

    def start_in(ci, s):
        r0 = rbase + ci * RCHUNK
        pltpu.async_copy(pred_hbm.at[b, pl.ds(r0, RCHUNK), :], pbufs[s],
                         isems[s])
        pltpu.async_copy(target_hbm.at[b, pl.ds(r0, RCHUNK), :], tbufs[s],
                         isems[s])

    def wait_in(s):
        pltpu.make_async_copy(
            pred_hbm.at[b, pl.ds(rbase, RCHUNK), :], pbufs[s], isems[s]).wait()
        pltpu.make_async_copy(
            target_hbm.at[b, pl.ds(rbase, RCHUNK), :], tbufs[s],
            isems[s]).wait()

    def wait_out(s):
        pltpu.make_async_copy(
            bins_hbm.at[pl.ds(0, WORDS)], wbufs[s], osems[s]).wait()

    start_in(0, 0)
    start_in(1, 1)

    @pl.loop(0, NCHUNK // 2)
    def ring(k):
        for s in range(2):
            ci = k * 2 + s
            wait_in(s)

            @pl.when(k > 0)
            def _():
                wait_out(s)

            pb, tb, wb = pbufs[s], tbufs[s], wbufs[s]

            @pl.loop(0, 16)
            def vec_body(j):
                c0 = j * 64
                for half in range(4):
                    rs = tuple(range(half * 4, half * 4 + 4))
                    ps, ts = {}, {}
                    for r in rs:
                        for u in range(4):
                            ps[r, u] = pb[r, pl.ds(c0 + u * L, L)]
                            ts[r, u] = tb[r, pl.ds(c0 + u * L, L)]
                    gs = {k2: jnp.abs(ps[k2] - ts[k2]) for k2 in ps}
                    bv = {k2: (gs[k2] * 10.0).astype(jnp.int32) for k2 in gs}
                    for r in rs:
                        for u in range(4):
                            plsc.addupdate_scatter(
                                hbufs[(r * 4 + u) % 8],
                                [lane16 + bv[r, u]], ones)
                    for r in rs:
                        word = (bv[r, 0] | (bv[r, 1] << 8)
                                | (bv[r, 2] << 16) | (bv[r, 3] << 24))
                        wb[pl.ds((r * 16 + j) * L, L)] = word

            woff = pl.multiple_of((wid * NCHUNK + ci) * WORDS, WORDS)
            pltpu.async_copy(wb, bins_hbm.at[pl.ds(woff, WORDS)], osems[s])

            @pl.when(ci + 2 < NCHUNK)
            def _():
                start_in(ci + 2, s)

    wait_out(0)
    wait_out(1)

    acc = hbufs[0][pl.ds(0, L)]
    for m in range(8):
        for i in range(16):
            if m == 0 and i == 0:
                continue
            acc = acc + hbufs[m][pl.ds(i * L, L)]
    wbuf0[pl.ds(0, L)] = acc
    pltpu.sync_copy(wbuf0.at[pl.ds(0, L)], hist_hbm.at[pl.ds(wid * L, L)])


@functools.partial(
    pl.kernel,
    out_type=jax.ShapeDtypeStruct((B, R, C), jnp.float32),
    mesh=_mesh,
    scratch_types=[
        pltpu.VMEM((NW * 16,), jnp.int32),
        pltpu.VMEM((WORDS,), jnp.int32),
        pltpu.VMEM((WORDS,), jnp.int32),
        pltpu.VMEM((RCHUNK, C), jnp.float32),
        pltpu.VMEM((RCHUNK, C), jnp.float32),
        pltpu.VMEM((16,), jnp.float32),
        pltpu.SemaphoreType.DMA,
        pltpu.SemaphoreType.DMA,
        pltpu.SemaphoreType.DMA,
        pltpu.SemaphoreType.DMA,
    ],
    compiler_params=_params,
)
def _pass2(hist_hbm, bins_hbm, out_hbm, histbuf, wordbuf0, wordbuf1,
           obuf0, obuf1, tabbuf, isem0, isem1, osem0, osem1):
    wid = lax.axis_index("s") * NC + lax.axis_index("c")
    b = wid // 2
    rbase = (wid % 2) * ROWS_W
    wordbufs, obufs = (wordbuf0, wordbuf1), (obuf0, obuf1)
    isems, osems = (isem0, isem1), (osem0, osem1)

    pltpu.sync_copy(hist_hbm, histbuf)
    counts = histbuf[pl.ds(0, L)]
    for wdx in range(1, NW):
        counts = counts + histbuf[pl.ds(wdx * L, L)]
    cf = counts.astype(jnp.float32)
    nz = counts > 0
    n = plsc.all_reduce_population_count(nz)
    nf = jnp.maximum(n.astype(jnp.float32), 1.0)
    acc = jnp.maximum(cf * 0.9, 1e-12)
    tabbuf[...] = jnp.where(nz, TOT / acc, 0.0) / nf

    def start_in(ci, s):
        woff = pl.multiple_of((wid * NCHUNK + ci) * WORDS, WORDS)
        pltpu.async_copy(bins_hbm.at[pl.ds(woff, WORDS)], wordbufs[s],
                         isems[s])

    def wait_in(s):
        pltpu.make_async_copy(
            bins_hbm.at[pl.ds(0, WORDS)], wordbufs[s], isems[s]).wait()

    def wait_out(s):
        pltpu.make_async_copy(
            out_hbm.at[b, pl.ds(rbase, RCHUNK), :], obufs[s], osems[s]).wait()

    start_in(0, 0)
    start_in(1, 1)

    @pl.loop(0, NCHUNK // 2)
    def ring(k):
        for s in range(2):
            ci = k * 2 + s
            wait_in(s)

            @pl.when(k > 0)
            def _():
                wait_out(s)

            wb, ob = wordbufs[s], obufs[s]

            @pl.loop(0, 16)
            def vec_body(j):
                c0 = j * 64
                for half in range(2):
                    rs = tuple(range(half * 8, half * 8 + 8))
                    words = {r: wb[pl.ds((r * 16 + j) * L, L)] for r in rs}
                    parts = {}
                    for r in rs:
                        w = words[r]
                        parts[r, 0] = w & 255
                        parts[r, 1] = (w >> 8) & 255
                        parts[r, 2] = (w >> 16) & 255
                        parts[r, 3] = lax.shift_right_logical(w, 24)
                    res = {}
                    for r in rs:
                        for u in range(4):
                            res[r, u] = plsc.load_gather(
                                tabbuf, [parts[r, u]])
                    for r in rs:
                        for u in range(4):
                            ob[r, pl.ds(c0 + u * L, L)] = res[r, u]

            r0 = rbase + ci * RCHUNK
            pltpu.async_copy(ob, out_hbm.at[b, pl.ds(r0, RCHUNK), :], osems[s])

            @pl.when(ci + 2 < NCHUNK)
            def _():
                start_in(ci + 2, s)

    wait_out(0)
    wait_out(1)


def kernel(pred, target):
    t = target.astype(jnp.float32)
    hist, bins = _pass1(pred, t)
    return _pass2(hist, bins)


# speedup vs baseline: 5.5415x; 1.0308x over previous
import functools

import jax
import jax.numpy as jnp
from jax import lax
from jax.experimental import pallas as pl
from jax.experimental.pallas import tpu as pltpu
from jax.experimental.pallas import tpu_sc as plsc

NC = 2
NS = 16
L = 16
NW = NC * NS
B, R, C = 16, 1024, 1024
N = B * R * C
ROWS_W = (B * R) // NW
RCHUNK = 8
NCHUNK = ROWS_W // RCHUNK
CHUNK = RCHUNK * C
WORDS = CHUNK // 4
NBUF = 4
TOT = float(R * C)

_mesh = plsc.VectorSubcoreMesh(core_axis_name="c", subcore_axis_name="s")
_params = pltpu.CompilerParams(
    needs_layout_passes=False, use_tc_tiling_on_sc=True)

_GDN = lax.GatherDimensionNumbers(
    offset_dims=(), collapsed_slice_dims=(0,), start_index_map=(0,))


def _tab_lookup(tab, idx):
    return lax.gather(tab, idx[:, None], dimension_numbers=_GDN,
                      slice_sizes=(1,),
                      mode=lax.GatherScatterMode.PROMISE_IN_BOUNDS)


@functools.partial(
    pl.kernel,
    out_type=(
        jax.ShapeDtypeStruct((NW * 16,), jnp.int32),
        jax.ShapeDtypeStruct((N // 4,), jnp.int32),
    ),
    mesh=_mesh,
    scratch_types=(
        [pltpu.VMEM((RCHUNK, C), jnp.float32) for _ in range(2 * NBUF)]
        + [pltpu.VMEM((WORDS,), jnp.int32) for _ in range(NBUF)]
        + [pltpu.VMEM((256,), jnp.int32) for _ in range(8)]
        + [pltpu.SemaphoreType.DMA for _ in range(2 * NBUF)]
    ),
    compiler_params=_params,
)
def _pass1(pred_hbm, target_hbm, hist_hbm, bins_hbm, *scratch):
    pbufs = scratch[0:NBUF]
    tbufs = scratch[NBUF:2 * NBUF]
    wbufs = scratch[2 * NBUF:3 * NBUF]
    hbufs = scratch[3 * NBUF:3 * NBUF + 8]
    isems = scratch[3 * NBUF + 8:4 * NBUF + 8]
    osems = scratch[4 * NBUF + 8:5 * NBUF + 8]

    wid = lax.axis_index("s") * NC + lax.axis_index("c")
    b = wid // 2
    rbase = (wid % 2) * ROWS_W
    lane16 = lax.iota(jnp.int32, L) * 16
    zeros = jnp.zeros((L,), jnp.int32)
    ones = jnp.ones((L,), jnp.int32)
    for m in range(8):
        for i in range(16):
            hbufs[m][pl.ds(i * L, L)] = zeros

    def start_in(ci, s):
        r0 = rbase + ci * RCHUNK
        pltpu.async_copy(pred_hbm.at[b, pl.ds(r0, RCHUNK), :], pbufs[s],
                         isems[s])
        pltpu.async_copy(target_hbm.at[b, pl.ds(r0, RCHUNK), :], tbufs[s],
                         isems[s])

    def wait_in(s):
        pltpu.make_async_copy(
            pred_hbm.at[b, pl.ds(rbase, RCHUNK), :], pbufs[s], isems[s]).wait()
        pltpu.make_async_copy(
            target_hbm.at[b, pl.ds(rbase, RCHUNK), :], tbufs[s],
            isems[s]).wait()

    def wait_out(s):
        pltpu.make_async_copy(
            bins_hbm.at[pl.ds(0, WORDS)], wbufs[s], osems[s]).wait()

    for s in range(NBUF):
        start_in(s, s)

    @pl.loop(0, NCHUNK // NBUF)
    def ring(k):
        for s in range(NBUF):
            ci = k * NBUF + s
            wait_in(s)

            @pl.when(k > 0)
            def _():
                wait_out(s)

            pb, tb, wb = pbufs[s], tbufs[s], wbufs[s]

            @pl.loop(0, 8)
            def vec_body(jq):
                for qh in range(2):
                    q = jq * 2 + qh
                    c0 = q * 64
                    for half in range(2):
                        rs = tuple(range(half * 4, half * 4 + 4))
                        ps, ts = {}, {}
                        for r in rs:
                            for u in range(4):
                                ps[r, u] = pb[r, pl.ds(c0 + u * L, L)]
                                ts[r, u] = tb[r, pl.ds(c0 + u * L, L)]
                        gs = {k2: jnp.abs(ps[k2] - ts[k2]) for k2 in ps}
                        bv = {k2: (gs[k2] * 10.0).astype(jnp.int32)
                              for k2 in gs}
                        for r in rs:
                            for u in range(4):
                                plsc.addupdate_scatter(
                                    hbufs[(r * 4 + u) % 8],
                                    [lane16 + bv[r, u]], ones)
                        for r in rs:
                            word = (bv[r, 0] | (bv[r, 1] << 8)
                                    | (bv[r, 2] << 16) | (bv[r, 3] << 24))
                            wb[pl.ds((r * 16 + q) * L, L)] = word

            woff = pl.multiple_of((wid * NCHUNK + ci) * WORDS, WORDS)
            pltpu.async_copy(wb, bins_hbm.at[pl.ds(woff, WORDS)], osems[s])

            @pl.when(ci + NBUF < NCHUNK)
            def _():
                start_in(ci + NBUF, s)

    for s in range(NBUF):
        wait_out(s)

    acc = hbufs[0][pl.ds(0, L)]
    for m in range(8):
        for i in range(16):
            if m == 0 and i == 0:
                continue
            acc = acc + hbufs[m][pl.ds(i * L, L)]
    wbufs[0][pl.ds(0, L)] = acc
    pltpu.sync_copy(wbufs[0].at[pl.ds(0, L)], hist_hbm.at[pl.ds(wid * L, L)])


@functools.partial(
    pl.kernel,
    out_type=jax.ShapeDtypeStruct((B, R, C), jnp.float32),
    mesh=_mesh,
    scratch_types=(
        [pltpu.VMEM((NW * 16,), jnp.int32)]
        + [pltpu.VMEM((WORDS,), jnp.int32) for _ in range(NBUF)]
        + [pltpu.VMEM((RCHUNK, C), jnp.float32) for _ in range(NBUF)]
        + [pltpu.SemaphoreType.DMA for _ in range(2 * NBUF)]
    ),
    compiler_params=_params,
)
def _pass2(hist_hbm, bins_hbm, out_hbm, *scratch):
    histbuf = scratch[0]
    wordbufs = scratch[1:1 + NBUF]
    obufs = scratch[1 + NBUF:1 + 2 * NBUF]
    isems = scratch[1 + 2 * NBUF:1 + 3 * NBUF]
    osems = scratch[1 + 3 * NBUF:1 + 4 * NBUF]

    wid = lax.axis_index("s") * NC + lax.axis_index("c")
    b = wid // 2
    rbase = (wid % 2) * ROWS_W

    pltpu.sync_copy(hist_hbm, histbuf)
    counts = histbuf[pl.ds(0, L)]
    for wdx in range(1, NW):
        counts = counts + histbuf[pl.ds(wdx * L, L)]
    cf = counts.astype(jnp.float32)
    nz = counts > 0
    n = plsc.all_reduce_population_count(nz)
    nf = jnp.maximum(n.astype(jnp.float32), 1.0)
    acc = jnp.maximum(cf * 0.9, 1e-12)
    tab = jnp.where(nz, TOT / acc, 0.0) / nf

    def start_in(ci, s):
        woff = pl.multiple_of((wid * NCHUNK + ci) * WORDS, WORDS)
        pltpu.async_copy(bins_hbm.at[pl.ds(woff, WORDS)], wordbufs[s],
                         isems[s])

    def wait_in(s):
        pltpu.make_async_copy(
            bins_hbm.at[pl.ds(0, WORDS)], wordbufs[s], isems[s]).wait()

    def wait_out(s):
        pltpu.make_async_copy(
            out_hbm.at[b, pl.ds(rbase, RCHUNK), :], obufs[s], osems[s]).wait()

    for s in range(NBUF):
        start_in(s, s)

    @pl.loop(0, NCHUNK // NBUF)
    def ring(k):
        for s in range(NBUF):
            ci = k * NBUF + s
            wait_in(s)

            @pl.when(k > 0)
            def _():
                wait_out(s)

            wb, ob = wordbufs[s], obufs[s]

            @pl.loop(0, 8)
            def vec_body(jq):
                for qh in range(2):
                    q = jq * 2 + qh
                    c0 = q * 64
                    rs = tuple(range(8))
                    words = {r: wb[pl.ds((r * 16 + q) * L, L)] for r in rs}
                    parts = {}
                    for r in rs:
                        w = words[r]
                        parts[r, 0] = w & 255
                        parts[r, 1] = (w >> 8) & 255
                        parts[r, 2] = (w >> 16) & 255
                        parts[r, 3] = lax.shift_right_logical(w, 24)
                    res = {}
                    for r in rs:
                        for u in range(4):
                            res[r, u] = _tab_lookup(tab, parts[r, u])
                    for r in rs:
                        for u in range(4):
                            ob[r, pl.ds(c0 + u * L, L)] = res[r, u]

            r0 = rbase + ci * RCHUNK
            pltpu.async_copy(ob, out_hbm.at[b, pl.ds(r0, RCHUNK), :], osems[s])

            @pl.when(ci + NBUF < NCHUNK)
            def _():
                start_in(ci + NBUF, s)

    for s in range(NBUF):
        wait_out(s)


def kernel(pred, target):
    t = target.astype(jnp.float32)
    hist, bins = _pass1(pred, t)
    return _pass2(hist, bins)


# speedup vs baseline: 5.7582x; 1.0391x over previous
import functools

import jax
import jax.numpy as jnp
from jax import lax
from jax.experimental import pallas as pl
from jax.experimental.pallas import tpu as pltpu
from jax.experimental.pallas import tpu_sc as plsc

NC = 2
NS = 16
L = 16
NW = NC * NS
B, R, C = 16, 1024, 1024
N = B * R * C
ROWS_W = (B * R) // NW
RCHUNK = 16
NCHUNK = ROWS_W // RCHUNK
CHUNK = RCHUNK * C
WORDS = CHUNK // 4
NBUF = 2
TOT = float(R * C)

_mesh = plsc.VectorSubcoreMesh(core_axis_name="c", subcore_axis_name="s")
_params = pltpu.CompilerParams(
    needs_layout_passes=False, use_tc_tiling_on_sc=True)

_GDN = lax.GatherDimensionNumbers(
    offset_dims=(), collapsed_slice_dims=(0,), start_index_map=(0,))


def _tab_lookup(tab, idx):
    return lax.gather(tab, idx[:, None], dimension_numbers=_GDN,
                      slice_sizes=(1,),
                      mode=lax.GatherScatterMode.PROMISE_IN_BOUNDS)


@functools.partial(
    pl.kernel,
    out_type=(
        jax.ShapeDtypeStruct((NW * 16,), jnp.int32),
        jax.ShapeDtypeStruct((N // 4,), jnp.int32),
    ),
    mesh=_mesh,
    scratch_types=(
        [pltpu.VMEM((RCHUNK, C), jnp.float32) for _ in range(2 * NBUF)]
        + [pltpu.VMEM((WORDS,), jnp.int32) for _ in range(NBUF)]
        + [pltpu.VMEM((256,), jnp.int32) for _ in range(8)]
        + [pltpu.SemaphoreType.DMA for _ in range(2 * NBUF)]
    ),
    compiler_params=_params,
)
def _pass1(pred_hbm, target_hbm, hist_hbm, bins_hbm, *scratch):
    pbufs = scratch[0:NBUF]
    tbufs = scratch[NBUF:2 * NBUF]
    wbufs = scratch[2 * NBUF:3 * NBUF]
    hbufs = scratch[3 * NBUF:3 * NBUF + 8]
    isems = scratch[3 * NBUF + 8:4 * NBUF + 8]
    osems = scratch[4 * NBUF + 8:5 * NBUF + 8]

    wid = lax.axis_index("s") * NC + lax.axis_index("c")
    b = wid // 2
    rbase = (wid % 2) * ROWS_W
    lane16 = lax.iota(jnp.int32, L) * 16
    zeros = jnp.zeros((L,), jnp.int32)
    ones = jnp.ones((L,), jnp.int32)
    for m in range(8):
        for i in range(16):
            hbufs[m][pl.ds(i * L, L)] = zeros

    def start_in(ci, s):
        r0 = rbase + ci * RCHUNK
        pltpu.async_copy(pred_hbm.at[b, pl.ds(r0, RCHUNK), :], pbufs[s],
                         isems[s])
        pltpu.async_copy(target_hbm.at[b, pl.ds(r0, RCHUNK), :], tbufs[s],
                         isems[s])

    def wait_in(s):
        pltpu.make_async_copy(
            pred_hbm.at[b, pl.ds(rbase, RCHUNK), :], pbufs[s], isems[s]).wait()
        pltpu.make_async_copy(
            target_hbm.at[b, pl.ds(rbase, RCHUNK), :], tbufs[s],
            isems[s]).wait()

    def wait_out(s):
        pltpu.make_async_copy(
            bins_hbm.at[pl.ds(0, WORDS)], wbufs[s], osems[s]).wait()

    for s in range(NBUF):
        start_in(s, s)

    @pl.loop(0, NCHUNK // NBUF)
    def ring(k):
        for s in range(NBUF):
            ci = k * NBUF + s
            wait_in(s)

            @pl.when(k > 0)
            def _():
                wait_out(s)

            pb, tb, wb = pbufs[s], tbufs[s], wbufs[s]

            @pl.loop(0, 16)
            def vec_body(q):
                if True:
                    c0 = q * 64
                    for half in range(RCHUNK // 4):
                        rs = tuple(range(half * 4, half * 4 + 4))
                        ps, ts = {}, {}
                        for r in rs:
                            for u in range(4):
                                ps[r, u] = pb[r, pl.ds(c0 + u * L, L)]
                                ts[r, u] = tb[r, pl.ds(c0 + u * L, L)]
                        gs = {k2: jnp.abs(ps[k2] - ts[k2]) for k2 in ps}
                        bv = {k2: (gs[k2] * 10.0).astype(jnp.int32)
                              for k2 in gs}
                        for r in rs:
                            for u in range(4):
                                plsc.addupdate_scatter(
                                    hbufs[(r * 4 + u) % 8],
                                    [lane16 + bv[r, u]], ones)
                        for r in rs:
                            word = (bv[r, 0] | (bv[r, 1] << 8)
                                    | (bv[r, 2] << 16) | (bv[r, 3] << 24))
                            wb[pl.ds((r * 16 + q) * L, L)] = word

            woff = pl.multiple_of((wid * NCHUNK + ci) * WORDS, WORDS)
            pltpu.async_copy(wb, bins_hbm.at[pl.ds(woff, WORDS)], osems[s])

            @pl.when(ci + NBUF < NCHUNK)
            def _():
                start_in(ci + NBUF, s)

    for s in range(NBUF):
        wait_out(s)

    acc = hbufs[0][pl.ds(0, L)]
    for m in range(8):
        for i in range(16):
            if m == 0 and i == 0:
                continue
            acc = acc + hbufs[m][pl.ds(i * L, L)]
    wbufs[0][pl.ds(0, L)] = acc
    pltpu.sync_copy(wbufs[0].at[pl.ds(0, L)], hist_hbm.at[pl.ds(wid * L, L)])


@functools.partial(
    pl.kernel,
    out_type=jax.ShapeDtypeStruct((B, R, C), jnp.float32),
    mesh=_mesh,
    scratch_types=(
        [pltpu.VMEM((NW * 16,), jnp.int32)]
        + [pltpu.VMEM((WORDS,), jnp.int32) for _ in range(NBUF)]
        + [pltpu.VMEM((RCHUNK, C), jnp.float32) for _ in range(NBUF)]
        + [pltpu.SemaphoreType.DMA for _ in range(2 * NBUF)]
    ),
    compiler_params=_params,
)
def _pass2(hist_hbm, bins_hbm, out_hbm, *scratch):
    histbuf = scratch[0]
    wordbufs = scratch[1:1 + NBUF]
    obufs = scratch[1 + NBUF:1 + 2 * NBUF]
    isems = scratch[1 + 2 * NBUF:1 + 3 * NBUF]
    osems = scratch[1 + 3 * NBUF:1 + 4 * NBUF]

    wid = lax.axis_index("s") * NC + lax.axis_index("c")
    b = wid // 2
    rbase = (wid % 2) * ROWS_W

    pltpu.sync_copy(hist_hbm, histbuf)
    counts = histbuf[pl.ds(0, L)]
    for wdx in range(1, NW):
        counts = counts + histbuf[pl.ds(wdx * L, L)]
    cf = counts.astype(jnp.float32)
    nz = counts > 0
    n = plsc.all_reduce_population_count(nz)
    nf = jnp.maximum(n.astype(jnp.float32), 1.0)
    acc = jnp.maximum(cf * 0.9, 1e-12)
    tab = jnp.where(nz, TOT / acc, 0.0) / nf

    def start_in(ci, s):
        woff = pl.multiple_of((wid * NCHUNK + ci) * WORDS, WORDS)
        pltpu.async_copy(bins_hbm.at[pl.ds(woff, WORDS)], wordbufs[s],
                         isems[s])

    def wait_in(s):
        pltpu.make_async_copy(
            bins_hbm.at[pl.ds(0, WORDS)], wordbufs[s], isems[s]).wait()

    def wait_out(s):
        pltpu.make_async_copy(
            out_hbm.at[b, pl.ds(rbase, RCHUNK), :], obufs[s], osems[s]).wait()

    for s in range(NBUF):
        start_in(s, s)

    @pl.loop(0, NCHUNK // NBUF)
    def ring(k):
        for s in range(NBUF):
            ci = k * NBUF + s
            wait_in(s)

            @pl.when(k > 0)
            def _():
                wait_out(s)

            wb, ob = wordbufs[s], obufs[s]

            @pl.loop(0, 16)
            def vec_body(q):
                for half in range(RCHUNK // 8):
                    c0 = q * 64
                    rs = tuple(range(half * 8, half * 8 + 8))
                    words = {r: wb[pl.ds((r * 16 + q) * L, L)] for r in rs}
                    parts = {}
                    for r in rs:
                        w = words[r]
                        parts[r, 0] = w & 255
                        parts[r, 1] = (w >> 8) & 255
                        parts[r, 2] = (w >> 16) & 255
                        parts[r, 3] = lax.shift_right_logical(w, 24)
                    res = {}
                    for r in rs:
                        for u in range(4):
                            res[r, u] = _tab_lookup(tab, parts[r, u])
                    for r in rs:
                        for u in range(4):
                            ob[r, pl.ds(c0 + u * L, L)] = res[r, u]

            r0 = rbase + ci * RCHUNK
            pltpu.async_copy(ob, out_hbm.at[b, pl.ds(r0, RCHUNK), :], osems[s])

            @pl.when(ci + NBUF < NCHUNK)
            def _():
                start_in(ci + NBUF, s)

    for s in range(NBUF):
        wait_out(s)


def kernel(pred, target):
    t = target.astype(jnp.float32)
    hist, bins = _pass1(pred, t)
    return _pass2(hist, bins)


# speedup vs baseline: 7.5165x; 1.3054x over previous
import functools

import jax
import jax.numpy as jnp
from jax import lax
from jax.experimental import pallas as pl
from jax.experimental.pallas import tpu as pltpu
from jax.experimental.pallas import tpu_sc as plsc

NC = 2
NS = 16
L = 16
NW = NC * NS
B, R, C = 16, 1024, 1024
TB = 7
BSC = B - TB
ROWS_W = (BSC * R) // NW
RCHUNK = 16
NCHUNK = ROWS_W // RCHUNK
TROWS_W = (TB * R) // NW
TNCHUNK = TROWS_W // RCHUNK
CHUNK = RCHUNK * C
WORDS = CHUNK // 4
NBUF = 2
BR = 256
TOT = float(R * C)

_mesh = plsc.VectorSubcoreMesh(core_axis_name="c", subcore_axis_name="s")
_params = pltpu.CompilerParams(
    needs_layout_passes=False, use_tc_tiling_on_sc=True)

_GDN = lax.GatherDimensionNumbers(
    offset_dims=(), collapsed_slice_dims=(0,), start_index_map=(0,))


def _tab_lookup(tab, idx):
    return lax.gather(tab, idx[:, None], dimension_numbers=_GDN,
                      slice_sizes=(1,),
                      mode=lax.GatherScatterMode.PROMISE_IN_BOUNDS)


def _p1tc_body(p_ref, t_ref, hist_ref, bins_ref):
    i = pl.program_id(0)
    j = pl.program_id(1)
    g = jnp.abs(p_ref[...] - t_ref[...])
    bv = (g * 10.0).astype(jnp.int32)
    bins_ref[...] = (bv[:, :, 0:256] | (bv[:, :, 256:512] << 8)
                     | (bv[:, :, 512:768] << 16) | (bv[:, :, 768:1024] << 24))

    @pl.when(jnp.logical_and(i == 0, j == 0))
    def _():
        hist_ref[...] = jnp.zeros((8, 128), jnp.int32)

    lane = lax.broadcasted_iota(jnp.int32, (8, 128), 1)
    sub = lax.broadcasted_iota(jnp.int32, (8, 128), 0)
    hv = jnp.zeros((8, 128), jnp.int32)
    for q in range(10):
        cq = jnp.sum((bv == q).astype(jnp.int32))
        hv = hv + jnp.where(jnp.logical_and(sub == 0, lane == q), cq, 0)
    hist_ref[...] = hist_ref[...] + hv


_pass1_tc = pl.pallas_call(
    _p1tc_body,
    grid=(TB, R // BR),
    in_specs=[
        pl.BlockSpec((1, BR, C), lambda i, j: (BSC + i, j, 0)),
        pl.BlockSpec((1, BR, C), lambda i, j: (BSC + i, j, 0)),
    ],
    out_specs=[
        pl.BlockSpec((8, 128), lambda i, j: (0, 0)),
        pl.BlockSpec((1, BR, C // 4), lambda i, j: (i, j, 0)),
    ],
    out_shape=[
        jax.ShapeDtypeStruct((8, 128), jnp.int32),
        jax.ShapeDtypeStruct((TB, R, C // 4), jnp.int32),
    ],
)


@functools.partial(
    pl.kernel,
    out_type=(
        jax.ShapeDtypeStruct((NW * 16,), jnp.int32),
        jax.ShapeDtypeStruct((BSC * R * C // 4,), jnp.int32),
    ),
    mesh=_mesh,
    scratch_types=(
        [pltpu.VMEM((RCHUNK, C), jnp.float32) for _ in range(2 * NBUF)]
        + [pltpu.VMEM((WORDS,), jnp.int32) for _ in range(NBUF)]
        + [pltpu.VMEM((256,), jnp.int32) for _ in range(8)]
        + [pltpu.SemaphoreType.DMA for _ in range(2 * NBUF)]
    ),
    compiler_params=_params,
)
def _pass1(pred_hbm, target_hbm, hist_hbm, bins_hbm, *scratch):
    pbufs = scratch[0:NBUF]
    tbufs = scratch[NBUF:2 * NBUF]
    wbufs = scratch[2 * NBUF:3 * NBUF]
    hbufs = scratch[3 * NBUF:3 * NBUF + 8]
    isems = scratch[3 * NBUF + 8:4 * NBUF + 8]
    osems = scratch[4 * NBUF + 8:5 * NBUF + 8]

    wid = lax.axis_index("s") * NC + lax.axis_index("c")
    lane16 = lax.iota(jnp.int32, L) * 16
    zeros = jnp.zeros((L,), jnp.int32)
    ones = jnp.ones((L,), jnp.int32)
    for m in range(8):
        for i in range(16):
            hbufs[m][pl.ds(i * L, L)] = zeros

    def start_in(ci, s):
        row0 = (wid * NCHUNK + ci) * RCHUNK
        b = row0 // R
        r0 = row0 % R
        pltpu.async_copy(pred_hbm.at[b, pl.ds(r0, RCHUNK), :], pbufs[s],
                         isems[s])
        pltpu.async_copy(target_hbm.at[b, pl.ds(r0, RCHUNK), :], tbufs[s],
                         isems[s])

    def wait_in(s):
        pltpu.make_async_copy(
            pred_hbm.at[0, pl.ds(0, RCHUNK), :], pbufs[s], isems[s]).wait()
        pltpu.make_async_copy(
            target_hbm.at[0, pl.ds(0, RCHUNK), :], tbufs[s], isems[s]).wait()

    def wait_out(s):
        pltpu.make_async_copy(
            bins_hbm.at[pl.ds(0, WORDS)], wbufs[s], osems[s]).wait()

    for s in range(NBUF):
        start_in(s, s)

    @pl.loop(0, NCHUNK // NBUF)
    def ring(k):
        for s in range(NBUF):
            ci = k * NBUF + s
            wait_in(s)

            @pl.when(k > 0)
            def _():
                wait_out(s)

            pb, tb, wb = pbufs[s], tbufs[s], wbufs[s]

            @pl.loop(0, 16)
            def vec_body(q):
                c0 = q * 64
                for half in range(RCHUNK // 4):
                    rs = tuple(range(half * 4, half * 4 + 4))
                    ps, ts = {}, {}
                    for r in rs:
                        for u in range(4):
                            ps[r, u] = pb[r, pl.ds(c0 + u * L, L)]
                            ts[r, u] = tb[r, pl.ds(c0 + u * L, L)]
                    gs = {k2: jnp.abs(ps[k2] - ts[k2]) for k2 in ps}
                    bv = {k2: (gs[k2] * 10.0).astype(jnp.int32) for k2 in gs}
                    for r in rs:
                        for u in range(4):
                            plsc.addupdate_scatter(
                                hbufs[(r * 4 + u) % 8],
                                [lane16 + bv[r, u]], ones)
                    for r in rs:
                        word = (bv[r, 0] | (bv[r, 1] << 8)
                                | (bv[r, 2] << 16) | (bv[r, 3] << 24))
                        wb[pl.ds((r * 16 + q) * L, L)] = word

            woff = pl.multiple_of((wid * NCHUNK + ci) * WORDS, WORDS)
            pltpu.async_copy(wb, bins_hbm.at[pl.ds(woff, WORDS)], osems[s])

            @pl.when(ci + NBUF < NCHUNK)
            def _():
                start_in(ci + NBUF, s)

    for s in range(NBUF):
        wait_out(s)

    acc = hbufs[0][pl.ds(0, L)]
    for m in range(8):
        for i in range(16):
            if m == 0 and i == 0:
                continue
            acc = acc + hbufs[m][pl.ds(i * L, L)]
    wbufs[0][pl.ds(0, L)] = acc
    pltpu.sync_copy(wbufs[0].at[pl.ds(0, L)], hist_hbm.at[pl.ds(wid * L, L)])


@functools.partial(
    pl.kernel,
    out_type=jax.ShapeDtypeStruct((B, R, C), jnp.float32),
    mesh=_mesh,
    scratch_types=(
        [pltpu.VMEM((NW * 16,), jnp.int32), pltpu.VMEM((8, 128), jnp.int32)]
        + [pltpu.VMEM((WORDS,), jnp.int32) for _ in range(NBUF)]
        + [pltpu.VMEM((RCHUNK, C // 4), jnp.int32) for _ in range(NBUF)]
        + [pltpu.VMEM((RCHUNK, C), jnp.float32) for _ in range(NBUF)]
        + [pltpu.SemaphoreType.DMA for _ in range(2 * NBUF)]
    ),
    compiler_params=_params,
)
def _pass2(hist_hbm, histtc_hbm, bins_hbm, binstc_hbm, out_hbm, *scratch):
    histbuf = scratch[0]
    histtcbuf = scratch[1]
    wordbufs = scratch[2:2 + NBUF]
    wordtcbufs = scratch[2 + NBUF:2 + 2 * NBUF]
    obufs = scratch[2 + 2 * NBUF:2 + 3 * NBUF]
    isems = scratch[2 + 3 * NBUF:2 + 4 * NBUF]
    osems = scratch[2 + 4 * NBUF:2 + 5 * NBUF]

    wid = lax.axis_index("s") * NC + lax.axis_index("c")

    pltpu.sync_copy(hist_hbm, histbuf)
    pltpu.sync_copy(histtc_hbm, histtcbuf)
    counts = histtcbuf[0, pl.ds(0, L)]
    for wdx in range(NW):
        counts = counts + histbuf[pl.ds(wdx * L, L)]
    cf = counts.astype(jnp.float32)
    nz = counts > 0
    n = plsc.all_reduce_population_count(nz)
    nf = jnp.maximum(n.astype(jnp.float32), 1.0)
    acc = jnp.maximum(cf * 0.9, 1e-12)
    tab = jnp.where(nz, TOT / acc, 0.0) / nf

    def sc_row(ci):
        return (wid * NCHUNK + ci) * RCHUNK

    def tc_row(ci):
        return (wid * TNCHUNK + ci) * RCHUNK

    def start_in_sc(ci, s):
        woff = pl.multiple_of((wid * NCHUNK + ci) * WORDS, WORDS)
        pltpu.async_copy(bins_hbm.at[pl.ds(woff, WORDS)], wordbufs[s],
                         isems[s])

    def start_in_tc(ci, s):
        row = tc_row(ci)
        bt = row // R
        r0 = row % R
        pltpu.async_copy(binstc_hbm.at[bt, pl.ds(r0, RCHUNK), :],
                         wordtcbufs[s], isems[s])

    def wait_in_sc(s):
        pltpu.make_async_copy(
            bins_hbm.at[pl.ds(0, WORDS)], wordbufs[s], isems[s]).wait()

    def wait_in_tc(s):
        pltpu.make_async_copy(
            binstc_hbm.at[0, pl.ds(0, RCHUNK), :], wordtcbufs[s],
            isems[s]).wait()

    def wait_out(s):
        pltpu.make_async_copy(
            out_hbm.at[0, pl.ds(0, RCHUNK), :], obufs[s], osems[s]).wait()

    def store_out(row, s):
        b = row // R
        r0 = row % R
        pltpu.async_copy(obufs[s], out_hbm.at[b, pl.ds(r0, RCHUNK), :],
                         osems[s])

    for s in range(NBUF):
        start_in_sc(s, s)

    @pl.loop(0, NCHUNK // NBUF)
    def ring(k):
        for s in range(NBUF):
            ci = k * NBUF + s
            wait_in_sc(s)

            @pl.when(k > 0)
            def _():
                wait_out(s)

            wb, ob = wordbufs[s], obufs[s]

            @pl.loop(0, 16)
            def vec_body(q):
                for half in range(RCHUNK // 8):
                    c0 = q * 64
                    rs = tuple(range(half * 8, half * 8 + 8))
                    words = {r: wb[pl.ds((r * 16 + q) * L, L)] for r in rs}
                    parts = {}
                    for r in rs:
                        w = words[r]
                        parts[r, 0] = w & 255
                        parts[r, 1] = (w >> 8) & 255
                        parts[r, 2] = (w >> 16) & 255
                        parts[r, 3] = lax.shift_right_logical(w, 24)
                    res = {}
                    for r in rs:
                        for u in range(4):
                            res[r, u] = _tab_lookup(tab, parts[r, u])
                    for r in rs:
                        for u in range(4):
                            ob[r, pl.ds(c0 + u * L, L)] = res[r, u]

            store_out(sc_row(ci), s)

            @pl.when(ci + NBUF < NCHUNK)
            def _():
                start_in_sc(ci + NBUF, s)

    for s in range(NBUF):
        wait_out(s)

    for s in range(NBUF):
        start_in_tc(s, s)

    @pl.loop(0, TNCHUNK // NBUF)
    def ring_tc(k):
        for s in range(NBUF):
            ci = k * NBUF + s
            wait_in_tc(s)

            @pl.when(k > 0)
            def _():
                wait_out(s)

            wb, ob = wordtcbufs[s], obufs[s]

            @pl.loop(0, 16)
            def vec_body(q):
                for half in range(RCHUNK // 8):
                    c16 = q * 16
                    rs = tuple(range(half * 8, half * 8 + 8))
                    words = {r: wb[r, pl.ds(c16, L)] for r in rs}
                    parts = {}
                    for r in rs:
                        w = words[r]
                        parts[r, 0] = w & 255
                        parts[r, 1] = (w >> 8) & 255
                        parts[r, 2] = (w >> 16) & 255
                        parts[r, 3] = lax.shift_right_logical(w, 24)
                    res = {}
                    for r in rs:
                        for u in range(4):
                            res[r, u] = _tab_lookup(tab, parts[r, u])
                    for r in rs:
                        for u in range(4):
                            ob[r, pl.ds(u * 256 + c16, L)] = res[r, u]

            store_out(BSC * R + tc_row(ci), s)

            @pl.when(ci + NBUF < TNCHUNK)
            def _():
                start_in_tc(ci + NBUF, s)

    for s in range(NBUF):
        wait_out(s)


def kernel(pred, target):
    t = target.astype(jnp.float32)
    hist_sc, bins_sc = _pass1(pred, t)
    hist_tc, bins_tc = _pass1_tc(pred, t)
    return _pass2(hist_sc, hist_tc, bins_sc, bins_tc)
